# baseline (speedup 1.0000x reference)
import functools

import jax
import jax.numpy as jnp
from jax import lax
from jax.experimental import pallas as pl
from jax.experimental.pallas import tpu as pltpu
from jax.experimental.pallas import tpu_sc as plsc

N = 100000
E = 128
G = 1024

NTILES = 32
CH = 3136
NPAD = NTILES * CH
NVR = CH // 16
TAILV = (N - (NTILES - 1) * CH) // 16
BN = 50176
NEG = -1.0e30


BROWS = BN // 128


def _logits_body(h_ref, w_ref, b_ref, out_ref, m_ref):
    pid = pl.program_id(0)
    hb = jnp.reshape(h_ref[...], (BROWS, 128, E))
    v = jnp.sum(hb * w_ref[...], axis=2) + b_ref[...]
    rows = (pid * BN
            + lax.broadcasted_iota(jnp.int32, (BROWS, 128), 0) * 128
            + lax.broadcasted_iota(jnp.int32, (BROWS, 128), 1))
    v = jnp.where(rows < N, v, NEG)
    out_ref[...] = v

    @pl.when(pid == 0)
    def _():
        m_ref[...] = jnp.full((1, 128), NEG, jnp.float32)

    m_ref[...] = jnp.maximum(m_ref[...], jnp.max(v))


_logits_call = pl.pallas_call(
    _logits_body,
    grid=(NPAD // BN,),
    in_specs=[
        pl.BlockSpec((BN, E), lambda i: (i, 0)),
        pl.BlockSpec((1, 1, E), lambda i: (0, 0, 0)),
        pl.BlockSpec((1, 1), lambda i: (0, 0)),
    ],
    out_specs=[
        pl.BlockSpec((BROWS, 128), lambda i: (i, 0)),
        pl.BlockSpec((1, 128), lambda i: (0, 0)),
    ],
    out_shape=[
        jax.ShapeDtypeStruct((NPAD // 128, 128), jnp.float32),
        jax.ShapeDtypeStruct((1, 128), jnp.float32),
    ],
)


@functools.cache
def _build_seg_kernel():
    sc_mesh = plsc.VectorSubcoreMesh(
        core_axis_name="c", subcore_axis_name="s", num_cores=2, num_subcores=16
    )

    @functools.partial(
        pl.kernel,
        out_type=(
            jax.ShapeDtypeStruct((NTILES, G), jnp.float32),
            jax.ShapeDtypeStruct((NTILES, G), jnp.float32),
            jax.ShapeDtypeStruct((G,), jnp.float32),
            jax.ShapeDtypeStruct((G,), jnp.int32),
        ),
        mesh=sc_mesh,
        compiler_params=pltpu.CompilerParams(needs_layout_passes=False),
        scratch_types=[
            pltpu.VMEM((CH,), jnp.float32),
            pltpu.VMEM((CH,), jnp.int32),
            pltpu.VMEM((G,), jnp.float32),
            pltpu.VMEM((G,), jnp.float32),
            pltpu.VMEM((G,), jnp.float32),
            pltpu.VMEM((G,), jnp.float32),
            pltpu.VMEM((128,), jnp.float32),
            pltpu.VMEM((G // NTILES,), jnp.int32),
            pltpu.VMEM((G // NTILES,), jnp.float32),
            pltpu.VMEM((G // NTILES,), jnp.int32),
            pltpu.SemaphoreType.DMA,
            pltpu.SemaphoreType.DMA,
            pltpu.SemaphoreType.DMA,
            pltpu.SemaphoreType.DMA,
        ],
    )
    def seg_kernel(logits_hbm, seg_hbm, act_hbm, m_hbm,
                   sp_out, tp_out, la_out, sa_out,
                   lg_v, sg_v, sacc, tacc, sacc2, tacc2, m_v, act_v, la_v, sa_v,
                   sem_in, sem_act, sem_g, sem_out):
        wid = lax.axis_index("s") * 2 + lax.axis_index("c")
        base = wid * CH
        abase = wid * (G // NTILES)
        cp_lg = pltpu.async_copy(logits_hbm.at[pl.ds(base, CH)], lg_v, sem_in)
        cp_m = pltpu.async_copy(m_hbm.at[0], m_v, sem_in)
        cp_act = pltpu.async_copy(
            act_hbm.at[pl.ds(abase, G // NTILES)], act_v, sem_act)

        zero16i = jnp.zeros((16,), jnp.int32)

        @pl.when(wid < NTILES - 1)
        def _():
            pltpu.sync_copy(seg_hbm.at[pl.ds(base, CH)], sg_v)

        @pl.when(wid == NTILES - 1)
        def _():
            for i in range(TAILV, NVR):
                sg_v[pl.ds(i * 16, 16)] = zero16i
            pltpu.sync_copy(
                seg_hbm.at[pl.ds((NTILES - 1) * CH, TAILV * 16)],
                sg_v.at[pl.ds(0, TAILV * 16)])

        zero16 = jnp.zeros((16,), jnp.float32)
        for i in range(G // 16):
            sacc[pl.ds(i * 16, 16)] = zero16
            tacc[pl.ds(i * 16, 16)] = zero16
            sacc2[pl.ds(i * 16, 16)] = zero16
            tacc2[pl.ds(i * 16, 16)] = zero16

        cp_act.wait()
        g_la = pltpu.async_copy(logits_hbm.at[act_v], la_v, sem_g)
        g_sa = pltpu.async_copy(seg_hbm.at[act_v], sa_v, sem_g)
        cp_lg.wait()
        cp_m.wait()

        m16 = m_v[pl.ds(0, 16)]

        @plsc.parallel_loop(0, NVR, step=2, unroll=7)
        def _(i):
            off = pl.multiple_of(i * 16, 16)
            for j in range(2):
                l = lg_v[pl.ds(off + j * 16, 16)]
                idx = sg_v[pl.ds(off + j * 16, 16)]
                sh = l - m16
                e = jnp.exp(sh)
                sa_t = sacc if j == 0 else sacc2
                ta_t = tacc if j == 0 else tacc2
                plsc.addupdate_scatter(sa_t, [idx], e)
                plsc.addupdate_scatter(ta_t, [idx], e * sh)

        for i in range(G // 16):
            sl = pl.ds(i * 16, 16)
            sacc[sl] = sacc[sl] + sacc2[sl]
            tacc[sl] = tacc[sl] + tacc2[sl]

        o_sp = pltpu.async_copy(sacc, sp_out.at[wid], sem_out)
        o_tp = pltpu.async_copy(tacc, tp_out.at[wid], sem_out)
        g_la.wait()
        g_sa.wait()
        o_la = pltpu.async_copy(
            la_v, la_out.at[pl.ds(abase, G // NTILES)], sem_out)
        o_sa = pltpu.async_copy(
            sa_v, sa_out.at[pl.ds(abase, G // NTILES)], sem_out)
        o_sp.wait()
        o_tp.wait()
        o_la.wait()
        o_sa.wait()

    return seg_kernel


def _fin_body(sp_ref, tp_ref, la_ref, sa_ref, m_ref, lp_ref, ent_ref):
    ones = jnp.full((1, NTILES), 1.0, jnp.float32)
    s = jnp.dot(ones, sp_ref[...], preferred_element_type=jnp.float32)
    t = jnp.dot(ones, tp_ref[...], preferred_element_type=jnp.float32)
    pos = s > 0.0
    safe_s = jnp.where(pos, s, 1.0)
    ls = jnp.log(safe_s)
    ent_g = jnp.where(pos, ls - t / safe_s, 0.0)
    ent_ref[...] = jnp.sum(ent_g, keepdims=True)[:, :1] * (1.0 / G)
    oht = (lax.broadcasted_iota(jnp.int32, (G, G), 0)
           == sa_ref[...]).astype(jnp.float32)
    ls_sel = jnp.dot(ls, oht, preferred_element_type=jnp.float32)
    lp_ref[...] = la_ref[...] - m_ref[:, :1] - ls_sel


_fin_call = pl.pallas_call(
    _fin_body,
    out_shape=[
        jax.ShapeDtypeStruct((1, G), jnp.float32),
        jax.ShapeDtypeStruct((1, 1), jnp.float32),
    ],
)


def kernel(actions, h, batch_idx, W, b):
    logits2d, m = _logits_call(h, W.reshape(1, 1, E), b.reshape(1, 1))
    logits = logits2d.reshape(NPAD)
    sp, tp, la, sa = _build_seg_kernel()(logits, batch_idx, actions, m)
    lp, ent = _fin_call(sp, tp, la.reshape(1, G), sa.reshape(1, G), m)
    return (lp.reshape(G), ent.reshape(()))


# speedup vs baseline: 1.1631x; 1.1631x over previous
import functools

import jax
import jax.numpy as jnp
from jax import lax
from jax.experimental import pallas as pl
from jax.experimental.pallas import tpu as pltpu
from jax.experimental.pallas import tpu_sc as plsc

N = 100000
E = 128
G = 1024

NTILES = 32
CH = 3136
NPAD = NTILES * CH
NVR = CH // 16
TAILV = (N - (NTILES - 1) * CH) // 16
BN = 14336
NEG = -1.0e30


BROWS = BN // 128


def _logits_body(h_ref, w_ref, b_ref, out_ref, m_ref):
    pid = pl.program_id(0)
    hb = jnp.reshape(h_ref[...], (BROWS, 128, E))
    v = jnp.sum(hb * w_ref[...], axis=2) + b_ref[...]
    rows = (pid * BN
            + lax.broadcasted_iota(jnp.int32, (BROWS, 128), 0) * 128
            + lax.broadcasted_iota(jnp.int32, (BROWS, 128), 1))
    v = jnp.where(rows < N, v, NEG)
    out_ref[...] = v

    @pl.when(pid == 0)
    def _():
        m_ref[...] = jnp.full((1, 128), NEG, jnp.float32)

    m_ref[...] = jnp.maximum(m_ref[...], jnp.max(v))


_logits_call = pl.pallas_call(
    _logits_body,
    grid=(NPAD // BN,),
    in_specs=[
        pl.BlockSpec((BN, E), lambda i: (i, 0)),
        pl.BlockSpec((1, 1, E), lambda i: (0, 0, 0)),
        pl.BlockSpec((1, 1), lambda i: (0, 0)),
    ],
    out_specs=[
        pl.BlockSpec((BROWS, 128), lambda i: (i, 0)),
        pl.BlockSpec((1, 128), lambda i: (0, 0)),
    ],
    out_shape=[
        jax.ShapeDtypeStruct((NPAD // 128, 128), jnp.float32),
        jax.ShapeDtypeStruct((1, 128), jnp.float32),
    ],
)


@functools.cache
def _build_seg_kernel():
    sc_mesh = plsc.VectorSubcoreMesh(
        core_axis_name="c", subcore_axis_name="s", num_cores=2, num_subcores=16
    )

    @functools.partial(
        pl.kernel,
        out_type=(
            jax.ShapeDtypeStruct((NTILES, G), jnp.float32),
            jax.ShapeDtypeStruct((NTILES, G), jnp.float32),
            jax.ShapeDtypeStruct((G,), jnp.float32),
            jax.ShapeDtypeStruct((G,), jnp.int32),
        ),
        mesh=sc_mesh,
        compiler_params=pltpu.CompilerParams(needs_layout_passes=False),
        scratch_types=[
            pltpu.VMEM((CH,), jnp.float32),
            pltpu.VMEM((CH + 16,), jnp.int32),
            pltpu.VMEM((G,), jnp.float32),
            pltpu.VMEM((G,), jnp.float32),
            pltpu.VMEM((128,), jnp.float32),
            pltpu.VMEM((G // NTILES,), jnp.int32),
            pltpu.VMEM((G // NTILES,), jnp.float32),
            pltpu.VMEM((G // NTILES,), jnp.int32),
            pltpu.SemaphoreType.DMA,
            pltpu.SemaphoreType.DMA,
            pltpu.SemaphoreType.DMA,
            pltpu.SemaphoreType.DMA,
        ],
    )
    def seg_kernel(logits_hbm, seg_hbm, act_hbm, m_hbm,
                   sp_out, tp_out, la_out, sa_out,
                   lg_v, sg_v, sacc, tacc, m_v, act_v, la_v, sa_v,
                   sem_in, sem_act, sem_g, sem_out):
        wid = lax.axis_index("s") * 2 + lax.axis_index("c")
        base = wid * CH
        abase = wid * (G // NTILES)
        cp_lg = pltpu.async_copy(logits_hbm.at[pl.ds(base, CH)], lg_v, sem_in)
        cp_m = pltpu.async_copy(m_hbm.at[0], m_v, sem_in)
        cp_act = pltpu.async_copy(
            act_hbm.at[pl.ds(abase, G // NTILES)], act_v, sem_act)

        zero16i = jnp.zeros((16,), jnp.int32)

        @pl.when(wid < NTILES - 1)
        def _():
            pltpu.sync_copy(seg_hbm.at[pl.ds(base, CH)], sg_v.at[pl.ds(0, CH)])

        @pl.when(wid == NTILES - 1)
        def _():
            for i in range(TAILV, NVR):
                sg_v[pl.ds(i * 16, 16)] = zero16i
            pltpu.sync_copy(
                seg_hbm.at[pl.ds((NTILES - 1) * CH, TAILV * 16)],
                sg_v.at[pl.ds(0, TAILV * 16)])

        sg_v[pl.ds(CH, 16)] = jnp.full((16,), -1, jnp.int32)

        zero16 = jnp.zeros((16,), jnp.float32)
        for i in range(G // 16):
            sacc[pl.ds(i * 16, 16)] = zero16
            tacc[pl.ds(i * 16, 16)] = zero16

        cp_act.wait()
        g_la = pltpu.async_copy(logits_hbm.at[act_v], la_v, sem_g)
        g_sa = pltpu.async_copy(seg_hbm.at[act_v], sa_v, sem_g)
        cp_lg.wait()
        cp_m.wait()

        m16 = m_v[pl.ds(0, 16)]
        iota16 = lax.iota(jnp.int32, 16)
        sidx = jnp.maximum(iota16 - 1, 0)
        neg1 = jnp.full((16,), -1, jnp.int32)
        zerof = jnp.zeros((16,), jnp.float32)

        @plsc.parallel_loop(0, NVR, step=1, unroll=14)
        def _(i):
            off = pl.multiple_of(i * 16, 16)
            l = lg_v[pl.ds(off, 16)]
            idx = sg_v[pl.ds(off, 16)]
            idxn = sg_v[pl.ds(off + 1, 16)]
            sh = l - m16
            e = jnp.exp(sh)
            c1 = plsc.cumsum(e)
            c2 = plsc.cumsum(e * sh)
            m_end = idx != idxn
            midx = jnp.where(m_end, iota16, neg1)
            sm = jnp.where(iota16 == 0, neg1, midx[sidx])
            p = plsc.cummax(sm)
            pc = jnp.maximum(p, 0)
            valid = p >= 0
            f1 = jnp.where(valid, c1[pc], zerof)
            f2 = jnp.where(valid, c2[pc], zerof)
            m_scat = m_end | (iota16 == 15)
            plsc.addupdate_scatter(sacc, [idx], c1 - f1, mask=m_scat)
            plsc.addupdate_scatter(tacc, [idx], c2 - f2, mask=m_scat)

        o_sp = pltpu.async_copy(sacc, sp_out.at[wid], sem_out)
        o_tp = pltpu.async_copy(tacc, tp_out.at[wid], sem_out)
        g_la.wait()
        g_sa.wait()
        o_la = pltpu.async_copy(
            la_v, la_out.at[pl.ds(abase, G // NTILES)], sem_out)
        o_sa = pltpu.async_copy(
            sa_v, sa_out.at[pl.ds(abase, G // NTILES)], sem_out)
        o_sp.wait()
        o_tp.wait()
        o_la.wait()
        o_sa.wait()

    return seg_kernel


def _fin_body(sp_ref, tp_ref, la_ref, sa_ref, m_ref, lp_ref, ent_ref):
    ones = jnp.full((1, NTILES), 1.0, jnp.float32)
    s = jnp.dot(ones, sp_ref[...], preferred_element_type=jnp.float32)
    t = jnp.dot(ones, tp_ref[...], preferred_element_type=jnp.float32)
    pos = s > 0.0
    safe_s = jnp.where(pos, s, 1.0)
    ls = jnp.log(safe_s)
    ent_g = jnp.where(pos, ls - t / safe_s, 0.0)
    ent_ref[...] = jnp.sum(ent_g, keepdims=True)[:, :1] * (1.0 / G)
    oht = (lax.broadcasted_iota(jnp.int32, (G, G), 0)
           == sa_ref[...]).astype(jnp.float32)
    ls_sel = jnp.dot(ls, oht, preferred_element_type=jnp.float32)
    lp_ref[...] = la_ref[...] - m_ref[:, :1] - ls_sel


_fin_call = pl.pallas_call(
    _fin_body,
    out_shape=[
        jax.ShapeDtypeStruct((1, G), jnp.float32),
        jax.ShapeDtypeStruct((1, 1), jnp.float32),
    ],
)


def kernel(actions, h, batch_idx, W, b):
    logits2d, m = _logits_call(h, W.reshape(1, 1, E), b.reshape(1, 1))
    logits = logits2d.reshape(NPAD)
    sp, tp, la, sa = _build_seg_kernel()(logits, batch_idx, actions, m)
    lp, ent = _fin_call(sp, tp, la.reshape(1, G), sa.reshape(1, G), m)
    return (lp.reshape(G), ent.reshape(()))
